# baseline (speedup 1.0000x reference)
import functools

import jax
import jax.numpy as jnp
from jax import lax
from jax.experimental import pallas as pl
from jax.experimental.pallas import tpu as pltpu
from jax.experimental.pallas import tpu_sc as plsc

D_MODEL = 64
B_TOTAL = 4096 * 50

_info = plsc.get_sparse_core_info()
_NC, _NS = _info.num_cores, _info.num_subcores
_NW = _NC * _NS
_B_PER_W = B_TOTAL // _NW
_CHUNK = 128
_N_CHUNKS = _B_PER_W // _CHUNK
_GROUP = 5
_G_ROWS = _GROUP * _CHUNK
_N_GROUPS = _N_CHUNKS // _GROUP

_mesh = plsc.VectorSubcoreMesh(core_axis_name="c", subcore_axis_name="s")


@functools.partial(
    pl.kernel,
    out_type=jax.ShapeDtypeStruct((B_TOTAL, D_MODEL), jnp.float32),
    mesh=_mesh,
    scratch_types=[
        pltpu.VMEM((_B_PER_W,), jnp.int32),
        pltpu.VMEM((2, _G_ROWS, D_MODEL), jnp.float32),
        pltpu.SemaphoreType.DMA,
        pltpu.SemaphoreType.DMA,
        pltpu.SemaphoreType.DMA,
    ],
    compiler_params=pltpu.CompilerParams(use_tc_tiling_on_sc=False),
)
def _gather_kernel(src_hbm, table_hbm, out_hbm, idx_v, rows_v, gsem,
                   osem0, osem1):
    wid = lax.axis_index("s") * _NC + lax.axis_index("c")
    row_base = wid * _B_PER_W

    pltpu.async_copy(src_hbm.at[pl.ds(row_base, _B_PER_W)], idx_v, gsem).wait()

    osems = (osem0, osem1)

    def out_copy(g, b):
        return pltpu.make_async_copy(
            rows_v.at[b],
            out_hbm.at[pl.ds(row_base + g * _G_ROWS, _G_ROWS)],
            osems[b])

    def fill(g, b):
        copies = [
            pltpu.async_copy(
                table_hbm.at[idx_v.at[pl.ds((g * _GROUP + j) * _CHUNK, _CHUNK)]],
                rows_v.at[b, pl.ds(j * _CHUNK, _CHUNK)],
                gsem)
            for j in range(_GROUP)
        ]
        for c in copies:
            c.wait()
        out_copy(g, b).start()

    fill(0, 0)
    fill(1, 1)

    def body(h, carry):
        g = 2 * h
        out_copy(g - 2, 0).wait()
        fill(g, 0)
        out_copy(g - 1, 1).wait()
        fill(g + 1, 1)
        return carry

    lax.fori_loop(1, _N_GROUPS // 2, body, 0)
    out_copy(_N_GROUPS - 2, 0).wait()
    out_copy(_N_GROUPS - 1, 1).wait()


def kernel(src, weight):
    flat = src.reshape(-1).astype(jnp.int32)
    out = _gather_kernel(flat, weight)
    return out.reshape(src.shape[0], src.shape[1], D_MODEL)
